# baseline (speedup 1.0000x reference)
import functools

import jax
import jax.numpy as jnp
from jax import lax
from jax.experimental import pallas as pl
from jax.experimental.pallas import tpu as pltpu
from jax.experimental.pallas import tpu_sc as plsc

B = 16384
F = 26
V = 100000
H = 32
FC = 256
ED = 128

FH = F * H
NC, NS = 2, 16
NW = NC * NS
PAIRS_W = FH // NW
VEC = 16
VLO = 49920
VHI = V - VLO
CHB = B // 2
SQ = CHB // 2


def _sc_gather_body(tables_hbm, obs_hbm, xt_hbm,
                    row_lo, row_hi, idxf, out_a, out_b,
                    sem_lo, sem_hi, sem_oa, sem_ob):
    cid = lax.axis_index("c")
    sid = lax.axis_index("s")
    wid = sid * NC + cid
    p0 = wid * PAIRS_W
    f0 = lax.shift_right_logical(p0, 5)
    h0 = lax.bitwise_and(p0, 31)
    pltpu.async_copy(tables_hbm.at[f0, h0, pl.ds(0, VLO)], row_lo, sem_lo)
    pltpu.async_copy(tables_hbm.at[f0, h0, pl.ds(VLO, VHI)], row_hi, sem_hi)
    pltpu.async_copy(out_a, xt_hbm.at[p0, pl.ds(0, SQ)], sem_oa)
    pltpu.async_copy(out_b, xt_hbm.at[p0, pl.ds(SQ, SQ)], sem_ob)

    lane = lax.iota(jnp.int32, VEC)

    def pbody(i, prev_f):
        p = p0 + i
        f = lax.shift_right_logical(p, 5)
        h = lax.bitwise_and(p, 31)

        @pl.when(f != prev_f)
        def _stage_obs():
            pltpu.sync_copy(obs_hbm.at[f], idxf)

        def _prefetch(seg_lo, buf, sem):
            nxt = p + 1
            nf = lax.shift_right_logical(nxt, 5)
            nh = lax.bitwise_and(nxt, 31)
            if seg_lo:
                pltpu.async_copy(tables_hbm.at[nf, nh, pl.ds(0, VLO)], buf, sem)
            else:
                pltpu.async_copy(tables_hbm.at[nf, nh, pl.ds(VLO, VHI)], buf, sem)

        for bh in range(2):
            if bh == 0:
                pltpu.make_async_copy(
                    tables_hbm.at[f, h, pl.ds(0, VLO)], row_lo, sem_lo).wait()
            for sq in range(2):
                buf, sem = ((out_a, sem_oa), (out_b, sem_ob))[sq]
                base = bh * CHB + sq * SQ
                dst = xt_hbm.at[p, pl.ds(base, SQ)]
                pltpu.make_async_copy(buf, dst, sem).wait()

                @plsc.parallel_loop(0, SQ // VEC, unroll=8)
                def _lo_pass(j):
                    iv = idxf[pl.ds(base + j * VEC, VEC)]
                    ivc = jnp.minimum(iv, VLO - 1)
                    g = plsc.load_gather(row_lo, [ivc])
                    plsc.store_scatter(buf, [lane + j * VEC], g, mask=iv < VLO)

                if bh == 1 and sq == 1:
                    @pl.when(i + 1 < PAIRS_W)
                    def _pre_lo():
                        _prefetch(True, row_lo, sem_lo)
                if bh == 0 and sq == 0:
                    pltpu.make_async_copy(
                        tables_hbm.at[f, h, pl.ds(VLO, VHI)],
                        row_hi, sem_hi).wait()

                @plsc.parallel_loop(0, SQ // VEC, unroll=8)
                def _hi_pass(j):
                    iv = idxf[pl.ds(base + j * VEC, VEC)]
                    ivc = jnp.maximum(iv - VLO, 0)
                    g = plsc.load_gather(row_hi, [ivc])
                    plsc.store_scatter(buf, [lane + j * VEC], g, mask=iv >= VLO)

                pltpu.async_copy(buf, dst, sem)

                if bh == 1 and sq == 1:
                    @pl.when(i + 1 < PAIRS_W)
                    def _pre_hi():
                        _prefetch(False, row_hi, sem_hi)

        return f

    lax.fori_loop(0, PAIRS_W, pbody, jnp.int32(-1))
    pltpu.make_async_copy(
        out_a, xt_hbm.at[p0, pl.ds(0, SQ)], sem_oa).wait()
    pltpu.make_async_copy(
        out_b, xt_hbm.at[p0, pl.ds(SQ, SQ)], sem_ob).wait()


@jax.jit
def _sc_gather(tables_t, obs_t):
    mesh = plsc.VectorSubcoreMesh(
        core_axis_name="c", subcore_axis_name="s", num_cores=NC, num_subcores=NS
    )
    return pl.kernel(
        _sc_gather_body,
        out_type=jax.ShapeDtypeStruct((FH, B), jnp.float32),
        mesh=mesh,
        scratch_types=[
            pltpu.VMEM((VLO,), jnp.float32),
            pltpu.VMEM((VHI,), jnp.float32),
            pltpu.VMEM((B,), jnp.int32),
            pltpu.VMEM((SQ,), jnp.float32),
            pltpu.VMEM((SQ,), jnp.float32),
            pltpu.SemaphoreType.DMA,
            pltpu.SemaphoreType.DMA,
            pltpu.SemaphoreType.DMA,
            pltpu.SemaphoreType.DMA,
        ],
        compiler_params=pltpu.CompilerParams(needs_layout_passes=False),
    )(tables_t, obs_t)


def _mlp_body(xt_ref, w1t_ref, b1_ref, w2_ref, b2_ref, o_ref):
    yt = lax.dot_general(
        w1t_ref[...], xt_ref[...].astype(jnp.bfloat16),
        (((1,), (0,)), ((), ())),
        preferred_element_type=jnp.float32,
    )
    ht = jnp.maximum(yt + b1_ref[...], 0.0).astype(jnp.bfloat16)
    o = lax.dot_general(
        ht, w2_ref[...],
        (((0,), (0,)), ((), ())),
        preferred_element_type=jnp.float32,
    )
    o_ref[...] = o + b2_ref[...]


B_BLK = 4096


@jax.jit
def _tc_mlp(xt, W1T, b1c, W2, b2r):
    grid = (B // B_BLK,)
    return pl.pallas_call(
        _mlp_body,
        out_shape=jax.ShapeDtypeStruct((B, ED), jnp.float32),
        grid=grid,
        in_specs=[
            pl.BlockSpec((FH, B_BLK), lambda i: (0, i)),
            pl.BlockSpec((FC, FH), lambda i: (0, 0)),
            pl.BlockSpec((FC, 1), lambda i: (0, 0)),
            pl.BlockSpec((FC, ED), lambda i: (0, 0)),
            pl.BlockSpec((1, ED), lambda i: (0, 0)),
        ],
        out_specs=pl.BlockSpec((B_BLK, ED), lambda i: (i, 0)),
    )(xt, W1T, b1c, W2, b2r)


def kernel(obs, tables, W1, b1, W2, b2):
    tables_t = jnp.transpose(tables, (0, 2, 1))
    obs_t = jnp.transpose(obs.astype(jnp.int32))
    xt = _sc_gather(tables_t, obs_t)
    W1T = jnp.transpose(W1).astype(jnp.bfloat16)
    W2b = W2.astype(jnp.bfloat16)
    return _tc_mlp(xt, W1T, b1.reshape(FC, 1), W2b, b2.reshape(1, ED))


# speedup vs baseline: 1.1022x; 1.1022x over previous
import functools

import jax
import jax.numpy as jnp
from jax import lax
from jax.experimental import pallas as pl
from jax.experimental.pallas import tpu as pltpu
from jax.experimental.pallas import tpu_sc as plsc

B = 16384
F = 26
V = 100000
H = 32
FC = 256
ED = 128

FH = F * H
NC, NS = 2, 16
NW = NC * NS
PAIRS_W = FH // NW
VEC = 16
VLO = 49920
VHI = V - VLO
CHB = B // 2


def _sc_gather_body(tables_hbm, obs_hbm, xt_hbm,
                    row_lo, row_hi, idxf, out_v, sem_lo, sem_hi):
    cid = lax.axis_index("c")
    sid = lax.axis_index("s")
    wid = sid * NC + cid
    p0 = wid * PAIRS_W
    f0 = lax.shift_right_logical(p0, 5)
    h0 = lax.bitwise_and(p0, 31)
    pltpu.async_copy(tables_hbm.at[f0, h0, pl.ds(0, VLO)], row_lo, sem_lo)
    pltpu.async_copy(tables_hbm.at[f0, h0, pl.ds(VLO, VHI)], row_hi, sem_hi)

    lane = lax.iota(jnp.int32, VEC)

    def pbody(i, prev_f):
        p = p0 + i
        f = lax.shift_right_logical(p, 5)
        h = lax.bitwise_and(p, 31)

        @pl.when(f != prev_f)
        def _stage_obs():
            pltpu.sync_copy(obs_hbm.at[f], idxf)

        def _prefetch(seg_lo, buf, sem):
            nxt = p + 1
            nf = lax.shift_right_logical(nxt, 5)
            nh = lax.bitwise_and(nxt, 31)
            if seg_lo:
                pltpu.async_copy(tables_hbm.at[nf, nh, pl.ds(0, VLO)], buf, sem)
            else:
                pltpu.async_copy(tables_hbm.at[nf, nh, pl.ds(VLO, VHI)], buf, sem)

        for bh in range(2):
            if bh == 0:
                pltpu.make_async_copy(
                    tables_hbm.at[f, h, pl.ds(0, VLO)], row_lo, sem_lo).wait()

            @plsc.parallel_loop(0, CHB // VEC, unroll=8)
            def _lo_pass(j):
                iv = idxf[pl.ds(bh * CHB + j * VEC, VEC)]
                ivc = jnp.minimum(iv, VLO - 1)
                g = plsc.load_gather(row_lo, [ivc])
                plsc.store_scatter(out_v, [lane + j * VEC], g, mask=iv < VLO)

            if bh == 1:
                @pl.when(i + 1 < PAIRS_W)
                def _pre_lo():
                    _prefetch(True, row_lo, sem_lo)
            else:
                pltpu.make_async_copy(
                    tables_hbm.at[f, h, pl.ds(VLO, VHI)], row_hi, sem_hi).wait()

            @plsc.parallel_loop(0, CHB // VEC, unroll=8)
            def _hi_pass(j):
                iv = idxf[pl.ds(bh * CHB + j * VEC, VEC)]
                ivc = jnp.maximum(iv - VLO, 0)
                g = plsc.load_gather(row_hi, [ivc])
                plsc.store_scatter(out_v, [lane + j * VEC], g, mask=iv >= VLO)

            pltpu.sync_copy(out_v, xt_hbm.at[p, pl.ds(bh * CHB, CHB)])

            if bh == 1:
                @pl.when(i + 1 < PAIRS_W)
                def _pre_hi():
                    _prefetch(False, row_hi, sem_hi)

        return f

    lax.fori_loop(0, PAIRS_W, pbody, jnp.int32(-1))


@jax.jit
def _sc_gather(tables_t, obs_t):
    mesh = plsc.VectorSubcoreMesh(
        core_axis_name="c", subcore_axis_name="s", num_cores=NC, num_subcores=NS
    )
    return pl.kernel(
        _sc_gather_body,
        out_type=jax.ShapeDtypeStruct((FH, B), jnp.float32),
        mesh=mesh,
        scratch_types=[
            pltpu.VMEM((VLO,), jnp.float32),
            pltpu.VMEM((VHI,), jnp.float32),
            pltpu.VMEM((B,), jnp.int32),
            pltpu.VMEM((CHB,), jnp.float32),
            pltpu.SemaphoreType.DMA,
            pltpu.SemaphoreType.DMA,
        ],
        compiler_params=pltpu.CompilerParams(needs_layout_passes=False),
    )(tables_t, obs_t)


def _mlp_body(xt_ref, w1t_ref, b1_ref, w2_ref, b2_ref, o_ref):
    yt = lax.dot_general(
        w1t_ref[...], xt_ref[...].astype(jnp.bfloat16),
        (((1,), (0,)), ((), ())),
        preferred_element_type=jnp.float32,
    )
    ht = jnp.maximum(yt + b1_ref[...], 0.0).astype(jnp.bfloat16)
    o = lax.dot_general(
        ht, w2_ref[...],
        (((0,), (0,)), ((), ())),
        preferred_element_type=jnp.float32,
    )
    o_ref[...] = o + b2_ref[...]


B_BLK = 4096


@jax.jit
def _tc_mlp(xt, W1T, b1c, W2, b2r):
    grid = (B // B_BLK,)
    return pl.pallas_call(
        _mlp_body,
        out_shape=jax.ShapeDtypeStruct((B, ED), jnp.float32),
        grid=grid,
        in_specs=[
            pl.BlockSpec((FH, B_BLK), lambda i: (0, i)),
            pl.BlockSpec((FC, FH), lambda i: (0, 0)),
            pl.BlockSpec((FC, 1), lambda i: (0, 0)),
            pl.BlockSpec((FC, ED), lambda i: (0, 0)),
            pl.BlockSpec((1, ED), lambda i: (0, 0)),
        ],
        out_specs=pl.BlockSpec((B_BLK, ED), lambda i: (i, 0)),
    )(xt, W1T, b1c, W2, b2r)


def kernel(obs, tables, W1, b1, W2, b2):
    tables_t = jnp.transpose(tables, (0, 2, 1))
    obs_t = jnp.transpose(obs.astype(jnp.int32))
    xt = _sc_gather(tables_t, obs_t)
    W1T = jnp.transpose(W1).astype(jnp.bfloat16)
    W2b = W2.astype(jnp.bfloat16)
    return _tc_mlp(xt, W1T, b1.reshape(FC, 1), W2b, b2.reshape(1, ED))
